# baseline (speedup 1.0000x reference)
import functools

import jax
import jax.numpy as jnp
from jax import lax
from jax.experimental import pallas as pl
from jax.experimental.pallas import tpu as pltpu
from jax.experimental.pallas import tpu_sc as plsc

D = 768
LANES = 16
NC, NS = 2, 16
NW = NC * NS
BATCH = 4
SEQ = 2048
N = BATCH * SEQ
STRIPE = SEQ // NW
PER_W = BATCH * STRIPE
C = 32
NSLAB = PER_W // C
GROUPS = D // LANES


NBUF = 4


def _embed_body(x_hbm, tok_hbm, pos_hbm, out_hbm,
                  idx_v, mask_x, pos_idx, pos_buf, tok_buf,
                  sem_pos, sem_idx, sem_g0, sem_g1, sem_g2, sem_g3,
                  sem_s0, sem_s1, sem_s2, sem_s3):
    wid = lax.axis_index("s") * NC + lax.axis_index("c")
    l0 = wid * STRIPE

    lanes = lax.iota(jnp.int32, LANES)
    for j in range(STRIPE // LANES):
        pos_idx[pl.ds(j * LANES, LANES)] = lanes + (l0 + 1 + j * LANES)
    cp_pos = pltpu.async_copy(
        pos_hbm.at[pos_idx.at[pl.ds(0, C)]], pos_buf, sem_pos)
    idx_cps = [
        pltpu.async_copy(x_hbm.at[pl.ds(b * SEQ + l0, STRIPE)],
                         idx_v.at[pl.ds(b * STRIPE, STRIPE)], sem_idx)
        for b in range(BATCH)
    ]
    for cp in idx_cps:
        cp.wait()

    sem_g = (sem_g0, sem_g1, sem_g2, sem_g3)
    sem_s = (sem_s0, sem_s1, sem_s2, sem_s3)
    gathers = [None] * NBUF
    stores = [None] * NBUF

    def start_gather(s):
        p = s % NBUF
        h, b = s // BATCH, s % BATCH
        gathers[p] = pltpu.async_copy(
            tok_hbm.at[idx_v.at[pl.ds(b * STRIPE + h * C, C)]],
            tok_buf.at[p], sem_g[p])

    start_gather(0)
    start_gather(1)

    def mask_group(j):
        m = jnp.where(idx_v[pl.ds(j * LANES, LANES)] > 0, 1.0, 0.0)
        for r in range(LANES):
            mask_x[pl.ds((j * LANES + r) * LANES, LANES)] = m.at[
                jnp.full((LANES,), r, jnp.int32)].get(
                    mode="promise_in_bounds")
    plsc.parallel_loop(0, PER_W // LANES, 1, unroll=1)(mask_group)

    cp_pos.wait()

    for s in range(NSLAB):
        p = s % NBUF
        h, b = s // BATCH, s % BATCH
        if s + 2 < NSLAB:
            q = (s + 2) % NBUF
            if stores[q] is not None:
                stores[q].wait()
                stores[q] = None
            start_gather(s + 2)
        gathers[p].wait()

        def add_row(r):
            mrow = b * STRIPE + h * C + r
            mvec = mask_x[pl.ds(mrow * LANES, LANES)]
            for g in range(GROUPS):
                sl = pl.ds(g * LANES, LANES)
                plsc.addupdate(tok_buf.at[p, r, sl], pos_buf[r, sl] * mvec)

        if s == BATCH:
            cp_pos1a.wait()
            plsc.parallel_loop(0, C // 2, 1, unroll=4)(add_row)
            cp_pos1b.wait()
            plsc.parallel_loop(C // 2, C, 1, unroll=4)(add_row)
        else:
            plsc.parallel_loop(0, C, 1, unroll=8)(add_row)

        if s == BATCH - 1:
            cp_pos1a = pltpu.async_copy(
                pos_hbm.at[pos_idx.at[pl.ds(C, C // 2)]],
                pos_buf.at[pl.ds(0, C // 2)], sem_pos)
            cp_pos1b = pltpu.async_copy(
                pos_hbm.at[pos_idx.at[pl.ds(C + C // 2, C // 2)]],
                pos_buf.at[pl.ds(C // 2, C // 2)], sem_idx)

        out_row = b * SEQ + l0 + h * C
        stores[p] = pltpu.async_copy(
            tok_buf.at[p], out_hbm.at[pl.ds(out_row, C)], sem_s[p])

    for st in stores:
        if st is not None:
            st.wait()


def _build(interpret=False):
    return pl.kernel(
        _embed_body,
        out_type=jax.ShapeDtypeStruct((N, D), jnp.float32),
        mesh=plsc.VectorSubcoreMesh(
            core_axis_name="c", subcore_axis_name="s",
            num_cores=NC, num_subcores=NS),
        scratch_types=[
            pltpu.VMEM((PER_W,), jnp.int32),
            pltpu.VMEM((PER_W * LANES,), jnp.float32),
            pltpu.VMEM((STRIPE,), jnp.int32),
            pltpu.VMEM((C, D), jnp.float32),
            pltpu.VMEM((NBUF, C, D), jnp.float32),
            pltpu.SemaphoreType.DMA,
            pltpu.SemaphoreType.DMA,
            pltpu.SemaphoreType.DMA,
            pltpu.SemaphoreType.DMA,
            pltpu.SemaphoreType.DMA,
            pltpu.SemaphoreType.DMA,
            pltpu.SemaphoreType.DMA,
            pltpu.SemaphoreType.DMA,
            pltpu.SemaphoreType.DMA,
            pltpu.SemaphoreType.DMA,
        ],
        interpret=interpret,
    )


_embed_kernel = _build()


def kernel(x, token_table, pos_table):
    B, L = x.shape
    out = _embed_kernel(x.reshape(-1), token_table, pos_table)
    return out.reshape(B, L, D)


# speedup vs baseline: 1.0474x; 1.0474x over previous
import functools

import jax
import jax.numpy as jnp
from jax import lax
from jax.experimental import pallas as pl
from jax.experimental.pallas import tpu as pltpu
from jax.experimental.pallas import tpu_sc as plsc

D = 768
LANES = 16
NC, NS = 2, 16
NW = NC * NS
BATCH = 4
SEQ = 2048
N = BATCH * SEQ
STRIPE = SEQ // NW
PER_W = BATCH * STRIPE
C = 32
NSLAB = PER_W // C
GROUPS = D // LANES


NBUF = 4


def _embed_body(x_hbm, tok_hbm, pos_hbm, out_hbm,
                  idx_v, mask_x, pos_idx, pos_buf, tok_buf,
                  sem_pos, sem_idx, sem_g0, sem_g1, sem_g2, sem_g3,
                  sem_s0, sem_s1, sem_s2, sem_s3):
    wid = lax.axis_index("s") * NC + lax.axis_index("c")
    l0 = wid * STRIPE

    lanes = lax.iota(jnp.int32, LANES)
    for j in range(STRIPE // LANES):
        pos_idx[pl.ds(j * LANES, LANES)] = lanes + (l0 + 1 + j * LANES)
    cp_pos = pltpu.async_copy(
        pos_hbm.at[pos_idx.at[pl.ds(0, C)]], pos_buf, sem_pos)
    idx_cps = [
        pltpu.async_copy(x_hbm.at[pl.ds(b * SEQ + l0, STRIPE)],
                         idx_v.at[pl.ds(b * STRIPE, STRIPE)], sem_idx)
        for b in range(BATCH)
    ]
    for cp in idx_cps:
        cp.wait()

    sem_g = (sem_g0, sem_g1, sem_g2, sem_g3)
    sem_s = (sem_s0, sem_s1, sem_s2, sem_s3)
    gathers = [None] * NBUF
    stores = [None] * NBUF

    def start_gather(s):
        p = s % NBUF
        h, b = s // BATCH, s % BATCH
        gathers[p] = pltpu.async_copy(
            tok_hbm.at[idx_v.at[pl.ds(b * STRIPE + h * C, C)]],
            tok_buf.at[p], sem_g[p])

    start_gather(0)
    start_gather(1)

    def mask_group(j):
        m = jnp.where(idx_v[pl.ds(j * LANES, LANES)] > 0, 1.0, 0.0)
        for r in range(LANES):
            mask_x[pl.ds((j * LANES + r) * LANES, LANES)] = m.at[
                jnp.full((LANES,), r, jnp.int32)].get(
                    mode="promise_in_bounds")
    plsc.parallel_loop(0, PER_W // LANES, 1, unroll=1)(mask_group)

    cp_pos.wait()

    for s in range(NSLAB):
        p = s % NBUF
        h, b = s // BATCH, s % BATCH
        if s + 2 < NSLAB:
            q = (s + 2) % NBUF
            if stores[q] is not None:
                stores[q].wait()
                stores[q] = None
            start_gather(s + 2)
        if s == BATCH:
            cp_pos.wait()
        gathers[p].wait()

        def add_row(r):
            mrow = b * STRIPE + h * C + r
            mvec = mask_x[pl.ds(mrow * LANES, LANES)]
            for g in range(GROUPS):
                sl = pl.ds(g * LANES, LANES)
                plsc.addupdate(tok_buf.at[p, r, sl], pos_buf[r, sl] * mvec)

        plsc.parallel_loop(0, C, 1, unroll=8)(add_row)

        if s == BATCH - 1:
            cp_pos = pltpu.async_copy(
                pos_hbm.at[pos_idx.at[pl.ds(C, C)]], pos_buf, sem_pos,
                priority=1)

        out_row = b * SEQ + l0 + h * C
        stores[p] = pltpu.async_copy(
            tok_buf.at[p], out_hbm.at[pl.ds(out_row, C)], sem_s[p])

    for st in stores:
        if st is not None:
            st.wait()


def _build(interpret=False):
    return pl.kernel(
        _embed_body,
        out_type=jax.ShapeDtypeStruct((N, D), jnp.float32),
        mesh=plsc.VectorSubcoreMesh(
            core_axis_name="c", subcore_axis_name="s",
            num_cores=NC, num_subcores=NS),
        scratch_types=[
            pltpu.VMEM((PER_W,), jnp.int32),
            pltpu.VMEM((PER_W * LANES,), jnp.float32),
            pltpu.VMEM((STRIPE,), jnp.int32),
            pltpu.VMEM((C, D), jnp.float32),
            pltpu.VMEM((NBUF, C, D), jnp.float32),
            pltpu.SemaphoreType.DMA,
            pltpu.SemaphoreType.DMA,
            pltpu.SemaphoreType.DMA,
            pltpu.SemaphoreType.DMA,
            pltpu.SemaphoreType.DMA,
            pltpu.SemaphoreType.DMA,
            pltpu.SemaphoreType.DMA,
            pltpu.SemaphoreType.DMA,
            pltpu.SemaphoreType.DMA,
            pltpu.SemaphoreType.DMA,
        ],
        interpret=interpret,
    )


_embed_kernel = _build()


def kernel(x, token_table, pos_table):
    B, L = x.shape
    out = _embed_kernel(x.reshape(-1), token_table, pos_table)
    return out.reshape(B, L, D)
